# baseline (speedup 1.0000x reference)
import functools

import jax
import jax.numpy as jnp
from jax import lax
from jax.experimental import pallas as pl
from jax.experimental.pallas import tpu as pltpu
from jax.experimental.pallas import tpu_sc as plsc

N = 10000
E = 320000
D = 128
G = 64
NC, NS, L = 2, 16, 16
NW = NC * NS
NPAD = 10240
EPW = NPAD
EPAD = NW * EPW
CH = 128
NCHUNK = EPW // CH
RCH = 128
CPS = NPAD // RCH // NS
BLK = 1000
NBLK = N // BLK
W16 = 16

def _mesh():
    return plsc.VectorSubcoreMesh(
        core_axis_name="c", subcore_axis_name="s", num_cores=NC, num_subcores=NS)


NBUF = 2
SEG = 20
NSEG = NCHUNK // SEG


def _feat_body(g_hbm, eidx_hbm, zeros_hbm, part_hbm, acc, ebuf, rows,
               gs0, gs1, ss0, ss1):
    gs = (gs0, gs1)
    ss = (ss0, ss1)
    c = lax.axis_index("c")
    s = lax.axis_index("s")
    wid = s * NC + c

    pltpu.sync_copy(zeros_hbm, rows.at[0])
    for k in range(CPS):
        row0 = (s * CPS + k) * RCH
        pltpu.sync_copy(rows.at[0], acc.at[pl.ds(row0, RCH)])
    plsc.subcore_barrier()

    @pl.loop(0, NSEG)
    def _(seg):
        pltpu.sync_copy(eidx_hbm.at[wid, pl.ds(seg * SEG, SEG)], ebuf)
        for b in range(NBUF):
            pltpu.async_copy(g_hbm.at[ebuf.at[b, 0]], rows.at[b], gs[b])

        @pl.loop(0, SEG, step=NBUF)
        def _(q):
            for b in range(NBUF):
                j = q + b
                pltpu.make_async_copy(g_hbm.at[ebuf.at[j, 0]], rows.at[b], gs[b]).wait()
                pltpu.async_copy(rows.at[b], acc.at[ebuf.at[j, 1]], ss[b], add=True)

                @pl.when(j + NBUF < SEG)
                def _():
                    pltpu.make_async_copy(rows.at[b], acc.at[ebuf.at[j, 1]], ss[b]).wait()
                    pltpu.async_copy(g_hbm.at[ebuf.at[j + NBUF, 0]], rows.at[b], gs[b])

        for b in range(NBUF):
            pltpu.make_async_copy(rows.at[b], acc.at[ebuf.at[0, 1]], ss[b]).wait()

    plsc.subcore_barrier()
    for k in range(CPS):
        row0 = (s * CPS + k) * RCH
        pltpu.sync_copy(acc.at[pl.ds(row0, RCH)], part_hbm.at[c, pl.ds(row0, RCH)])


NDBUF = 4


def _deg_body(eidx_hbm, ones_hbm, zeros_hbm, part_hbm, acc, ebuf, ones_v,
              ss0, ss1, ss2, ss3):
    ss = (ss0, ss1, ss2, ss3)
    c = lax.axis_index("c")
    s = lax.axis_index("s")
    wid = s * NC + c

    pltpu.sync_copy(eidx_hbm.at[wid], ebuf)
    pltpu.sync_copy(zeros_hbm, ones_v)
    for k in range(CPS):
        row0 = (s * CPS + k) * RCH
        pltpu.sync_copy(ones_v, acc.at[pl.ds(row0, RCH)])
    pltpu.sync_copy(ones_hbm, ones_v)
    plsc.subcore_barrier()

    @pl.loop(0, NCHUNK, step=NDBUF)
    def _(g):
        for b in range(NDBUF):
            i = g + b

            @pl.when(g >= NDBUF)
            def _():
                pltpu.make_async_copy(ones_v, acc.at[ebuf.at[i, 1]], ss[b]).wait()

            pltpu.async_copy(ones_v, acc.at[ebuf.at[i, 1]], ss[b], add=True)

    for b in range(NDBUF):
        i = NCHUNK - NDBUF + b
        pltpu.make_async_copy(ones_v, acc.at[ebuf.at[i, 1]], ss[b]).wait()

    plsc.subcore_barrier()
    for k in range(CPS):
        row0 = (s * CPS + k) * RCH
        pltpu.sync_copy(acc.at[pl.ds(row0, RCH)], part_hbm.at[c, pl.ds(row0, RCH)])


@functools.cache
def _feat_kernel():
    return pl.kernel(
        _feat_body,
        out_type=jax.ShapeDtypeStruct((NC, NPAD, D), jnp.float32),
        mesh=_mesh(),
        scratch_types=[
            pltpu.VMEM_SHARED((NPAD, D), jnp.float32),
            pltpu.VMEM((SEG, 2, CH), jnp.int32),
            pltpu.VMEM((NBUF, CH, D), jnp.float32),
        ] + [pltpu.SemaphoreType.DMA] * (2 * NBUF),
    )


@functools.cache
def _deg_kernel():
    return pl.kernel(
        _deg_body,
        out_type=jax.ShapeDtypeStruct((NC, NPAD, D), jnp.float32),
        mesh=_mesh(),
        scratch_types=[
            pltpu.VMEM_SHARED((NPAD, D), jnp.float32),
            pltpu.VMEM((NCHUNK, 2, CH), jnp.int32),
            pltpu.VMEM((CH, D), jnp.float32),
        ] + [pltpu.SemaphoreType.DMA] * NDBUF,
    )


def _tc1_body(x_ref, w_ref, degp_ref, g_ref, dis_ref):
    deg = degp_ref[0, :, 0:1] + degp_ref[1, :, 0:1] + 1.0
    dis = lax.rsqrt(deg)
    h = jnp.dot(x_ref[...], w_ref[...], preferred_element_type=jnp.float32)
    g_ref[...] = h * dis
    dis_ref[...] = dis


def _tc2_body(g1_ref, part_ref, dis_ref, w2_ref, b1_ref, g2_ref):
    stot = part_ref[0] + part_ref[1]
    dis = dis_ref[...]
    z = dis * (stot + g1_ref[...]) + b1_ref[...]
    h = jnp.tanh(z)
    g2_ref[...] = jnp.dot(h, w2_ref[...], preferred_element_type=jnp.float32) * dis


def _tc3_body(g2_ref, part_ref, dis_ref, b2_ref, wl_ref, bl_ref, batch_ref,
              out_ref, pool_acc, cnt_acc):
    i = pl.program_id(0)
    stot = part_ref[0] + part_ref[1]
    dis = dis_ref[...]
    z = dis * (stot + g2_ref[...]) + b2_ref[...]
    b = batch_ref[0]
    iot = lax.broadcasted_iota(jnp.int32, (G, BLK), 0)
    oh = (b == iot).astype(jnp.float32)

    @pl.when(i == 0)
    def _():
        pool_acc[...] = jnp.zeros_like(pool_acc)
        cnt_acc[...] = jnp.zeros_like(cnt_acc)

    pool_acc[...] += jnp.dot(oh, z, preferred_element_type=jnp.float32)
    cnt_acc[...] += jnp.broadcast_to(
        jnp.sum(oh, axis=1, keepdims=True), (G, D))

    @pl.when(i == NBLK - 1)
    def _():
        pooled = pool_acc[...] / jnp.maximum(cnt_acc[...], 1.0)
        o = jnp.dot(pooled, wl_ref[...], preferred_element_type=jnp.float32)
        out_ref[...] = o[:, 0:1] + bl_ref[...]


def _tc1(x, w1p, degp):
    return pl.pallas_call(
        _tc1_body,
        grid=(NBLK,),
        in_specs=[
            pl.BlockSpec((BLK, D), lambda i: (i, 0)),
            pl.BlockSpec((D, D), lambda i: (0, 0)),
            pl.BlockSpec((NC, BLK, D), lambda i: (0, i, 0)),
        ],
        out_specs=[
            pl.BlockSpec((BLK, D), lambda i: (i, 0)),
            pl.BlockSpec((BLK, 1), lambda i: (i, 0)),
        ],
        out_shape=[
            jax.ShapeDtypeStruct((N, D), jnp.float32),
            jax.ShapeDtypeStruct((N, 1), jnp.float32),
        ],
    )(x, w1p, degp)


def _tc2(g1, part, dis, w2p, b1p):
    return pl.pallas_call(
        _tc2_body,
        grid=(NBLK,),
        in_specs=[
            pl.BlockSpec((BLK, D), lambda i: (i, 0)),
            pl.BlockSpec((NC, BLK, D), lambda i: (0, i, 0)),
            pl.BlockSpec((BLK, 1), lambda i: (i, 0)),
            pl.BlockSpec((D, D), lambda i: (0, 0)),
            pl.BlockSpec((1, D), lambda i: (0, 0)),
        ],
        out_specs=pl.BlockSpec((BLK, D), lambda i: (i, 0)),
        out_shape=jax.ShapeDtypeStruct((N, D), jnp.float32),
    )(g1, part, dis, w2p, b1p)


def _tc3(g2, part, dis, b2p, wlp, blp, batch3):
    return pl.pallas_call(
        _tc3_body,
        grid=(NBLK,),
        in_specs=[
            pl.BlockSpec((BLK, D), lambda i: (i, 0)),
            pl.BlockSpec((NC, BLK, D), lambda i: (0, i, 0)),
            pl.BlockSpec((BLK, 1), lambda i: (i, 0)),
            pl.BlockSpec((1, D), lambda i: (0, 0)),
            pl.BlockSpec((D, D), lambda i: (0, 0)),
            pl.BlockSpec((1, 1), lambda i: (0, 0)),
            pl.BlockSpec((1, 1, BLK), lambda i: (i, 0, 0)),
        ],
        out_specs=pl.BlockSpec((G, 1), lambda i: (0, 0)),
        out_shape=jax.ShapeDtypeStruct((G, 1), jnp.float32),
        scratch_shapes=[
            pltpu.VMEM((G, D), jnp.float32),
            pltpu.VMEM((G, D), jnp.float32),
        ],
    )(g2, part, dis, b2p, wlp, blp, batch3)


def kernel(x, edge_index, batch, W1, b1, W2, b2, Wl, bl):
    src = edge_index[0].astype(jnp.int32)
    dst = edge_index[1].astype(jnp.int32)
    pad = EPAD - E
    srcp = jnp.concatenate([src, jnp.zeros((pad,), jnp.int32)])
    dstp = jnp.concatenate([dst, jnp.full((pad,), NPAD - 1, jnp.int32)])
    woff = (jnp.arange(NW, dtype=jnp.int32) % NC) * N
    srcw = (srcp.reshape(NW, EPW) + woff[:, None]).reshape(NW, NCHUNK, CH)
    eidx = jnp.stack([srcw, dstp.reshape(NW, NCHUNK, CH)], axis=2)
    batch3 = batch.astype(jnp.int32).reshape(NBLK, 1, BLK)

    H = W1.shape[1]
    w1p = jnp.pad(W1, ((0, 0), (0, D - H)))
    w2p = jnp.pad(W2, ((0, D - H), (0, D - H)))
    b1p = jnp.pad(b1, (0, D - H)).reshape(1, D)
    b2p = jnp.pad(b2, (0, D - H)).reshape(1, D)
    wlp = jnp.pad(Wl, ((0, D - H), (0, D - 1)))
    blp = bl.reshape(1, 1)

    zerosD = jnp.zeros((RCH, D), jnp.float32)
    onesD = jnp.tile(jnp.eye(1, D, 0, dtype=jnp.float32), (CH, 1))

    degp = _deg_kernel()(eidx, onesD, zerosD)
    g1, dis = _tc1(x, w1p, degp)
    s1 = _feat_kernel()(jnp.concatenate([g1, g1]), eidx, zerosD)
    g2 = _tc2(g1, s1, dis, w2p, b1p)
    s2 = _feat_kernel()(jnp.concatenate([g2, g2]), eidx, zerosD)
    return _tc3(g2, s2, dis, b2p, wlp, blp, batch3)


# speedup vs baseline: 1.0981x; 1.0981x over previous
import functools

import jax
import jax.numpy as jnp
from jax import lax
from jax.experimental import pallas as pl
from jax.experimental.pallas import tpu as pltpu
from jax.experimental.pallas import tpu_sc as plsc

N = 10000
E = 320000
D = 128
G = 64
NC, NS, L = 2, 16, 16
NW = NC * NS
NPAD = 10240
EPW = NPAD
EPAD = NW * EPW
CH = 128
NCHUNK = EPW // CH
RCH = 128
CPS = NPAD // RCH // NS
BLK = 1000
NBLK = N // BLK
W16 = 16

def _mesh():
    return plsc.VectorSubcoreMesh(
        core_axis_name="c", subcore_axis_name="s", num_cores=NC, num_subcores=NS)


NBUF = 2
SEGF = 20
NSEGF = NCHUNK // SEGF
ZCH = NPAD // CH // NS


def _feat_body(g_hbm, eidx_hbm, zeros_hbm, part_hbm, acc, ebuf, rows,
               gs0, gs1, ss0, ss1):
    gs = (gs0, gs1)
    ss = (ss0, ss1)
    c = lax.axis_index("c")
    s = lax.axis_index("s")
    wid = s * NC + c

    pltpu.sync_copy(zeros_hbm, rows.at[0])
    for k in range(ZCH):
        row0 = (s * ZCH + k) * CH
        pltpu.sync_copy(rows.at[0], acc.at[pl.ds(row0, CH)])
    plsc.subcore_barrier()

    @pl.loop(0, NSEGF)
    def _(seg):
        pltpu.sync_copy(eidx_hbm.at[wid, pl.ds(seg * SEGF, SEGF)], ebuf)
        for b in range(NBUF):
            pltpu.async_copy(g_hbm.at[ebuf.at[b, 0]], rows.at[b], gs[b])

        @pl.loop(0, SEGF, step=NBUF)
        def _(q):
            for b in range(NBUF):
                j = q + b
                pltpu.make_async_copy(g_hbm.at[ebuf.at[j, 0]], rows.at[b], gs[b]).wait()
                pltpu.async_copy(rows.at[b], acc.at[ebuf.at[j, 1]], ss[b], add=True)

                @pl.when(j + NBUF < SEGF)
                def _():
                    pltpu.make_async_copy(rows.at[b], acc.at[ebuf.at[j, 1]], ss[b]).wait()
                    pltpu.async_copy(g_hbm.at[ebuf.at[j + NBUF, 0]], rows.at[b], gs[b])

        for b in range(NBUF):
            pltpu.make_async_copy(rows.at[b], acc.at[ebuf.at[0, 1]], ss[b]).wait()

    plsc.subcore_barrier()
    for k in range(CPS):
        row0 = (s * CPS + k) * RCH
        pltpu.sync_copy(acc.at[pl.ds(row0, RCH)], part_hbm.at[c, pl.ds(row0, RCH)])


NDBUF = 4


def _deg_body(eidx_hbm, ones_hbm, zeros_hbm, part_hbm, acc, ebuf, ones_v,
              ss0, ss1, ss2, ss3):
    ss = (ss0, ss1, ss2, ss3)
    c = lax.axis_index("c")
    s = lax.axis_index("s")
    wid = s * NC + c

    pltpu.sync_copy(eidx_hbm.at[wid], ebuf)
    pltpu.sync_copy(zeros_hbm, ones_v)
    for k in range(CPS):
        row0 = (s * CPS + k) * RCH
        pltpu.sync_copy(ones_v, acc.at[pl.ds(row0, RCH)])
    pltpu.sync_copy(ones_hbm, ones_v)
    plsc.subcore_barrier()

    @pl.loop(0, NCHUNK, step=NDBUF)
    def _(g):
        for b in range(NDBUF):
            i = g + b

            @pl.when(g >= NDBUF)
            def _():
                pltpu.make_async_copy(ones_v, acc.at[ebuf.at[i, 1]], ss[b]).wait()

            pltpu.async_copy(ones_v, acc.at[ebuf.at[i, 1]], ss[b], add=True)

    for b in range(NDBUF):
        i = NCHUNK - NDBUF + b
        pltpu.make_async_copy(ones_v, acc.at[ebuf.at[i, 1]], ss[b]).wait()

    plsc.subcore_barrier()
    for k in range(CPS):
        row0 = (s * CPS + k) * RCH
        pltpu.sync_copy(acc.at[pl.ds(row0, RCH)], part_hbm.at[c, pl.ds(row0, RCH)])


@functools.cache
def _feat_kernel():
    return pl.kernel(
        _feat_body,
        out_type=jax.ShapeDtypeStruct((NC, NPAD, D), jnp.float32),
        mesh=_mesh(),
        scratch_types=[
            pltpu.VMEM_SHARED((NPAD, D), jnp.float32),
            pltpu.VMEM((SEGF, 2, CH), jnp.int32),
            pltpu.VMEM((NBUF, CH, D), jnp.float32),
        ] + [pltpu.SemaphoreType.DMA] * (2 * NBUF),
    )


@functools.cache
def _deg_kernel():
    return pl.kernel(
        _deg_body,
        out_type=jax.ShapeDtypeStruct((NC, NPAD, D), jnp.float32),
        mesh=_mesh(),
        scratch_types=[
            pltpu.VMEM_SHARED((NPAD, D), jnp.float32),
            pltpu.VMEM((NCHUNK, 2, CH), jnp.int32),
            pltpu.VMEM((CH, D), jnp.float32),
        ] + [pltpu.SemaphoreType.DMA] * NDBUF,
    )


def _tc1_body(x_ref, w_ref, degp_ref, g_ref, dis_ref):
    deg = degp_ref[0, :, 0:1] + degp_ref[1, :, 0:1] + 1.0
    dis = lax.rsqrt(deg)
    h = jnp.dot(x_ref[...], w_ref[...], preferred_element_type=jnp.float32)
    g_ref[...] = h * dis
    dis_ref[...] = dis


def _tc2_body(g1_ref, part_ref, dis_ref, w2_ref, b1_ref, g2_ref):
    stot = part_ref[0] + part_ref[1]
    dis = dis_ref[...]
    z = dis * (stot + g1_ref[...]) + b1_ref[...]
    h = jnp.tanh(z)
    g2_ref[...] = jnp.dot(h, w2_ref[...], preferred_element_type=jnp.float32) * dis


def _tc3_body(g2_ref, part_ref, dis_ref, b2_ref, wl_ref, bl_ref, batch_ref,
              out_ref, pool_acc, cnt_acc):
    i = pl.program_id(0)
    stot = part_ref[0] + part_ref[1]
    dis = dis_ref[...]
    z = dis * (stot + g2_ref[...]) + b2_ref[...]
    b = batch_ref[0]
    iot = lax.broadcasted_iota(jnp.int32, (G, BLK), 0)
    oh = (b == iot).astype(jnp.float32)

    @pl.when(i == 0)
    def _():
        pool_acc[...] = jnp.zeros_like(pool_acc)
        cnt_acc[...] = jnp.zeros_like(cnt_acc)

    pool_acc[...] += jnp.dot(oh, z, preferred_element_type=jnp.float32)
    cnt_acc[...] += jnp.broadcast_to(
        jnp.sum(oh, axis=1, keepdims=True), (G, D))

    @pl.when(i == NBLK - 1)
    def _():
        pooled = pool_acc[...] / jnp.maximum(cnt_acc[...], 1.0)
        o = jnp.dot(pooled, wl_ref[...], preferred_element_type=jnp.float32)
        out_ref[...] = o[:, 0:1] + bl_ref[...]


def _tc1(x, w1p, degp):
    return pl.pallas_call(
        _tc1_body,
        grid=(NBLK,),
        in_specs=[
            pl.BlockSpec((BLK, D), lambda i: (i, 0)),
            pl.BlockSpec((D, D), lambda i: (0, 0)),
            pl.BlockSpec((NC, BLK, D), lambda i: (0, i, 0)),
        ],
        out_specs=[
            pl.BlockSpec((BLK, D), lambda i: (i, 0)),
            pl.BlockSpec((BLK, 1), lambda i: (i, 0)),
        ],
        out_shape=[
            jax.ShapeDtypeStruct((N, D), jnp.float32),
            jax.ShapeDtypeStruct((N, 1), jnp.float32),
        ],
    )(x, w1p, degp)


def _tc2(g1, part, dis, w2p, b1p):
    return pl.pallas_call(
        _tc2_body,
        grid=(NBLK,),
        in_specs=[
            pl.BlockSpec((BLK, D), lambda i: (i, 0)),
            pl.BlockSpec((NC, BLK, D), lambda i: (0, i, 0)),
            pl.BlockSpec((BLK, 1), lambda i: (i, 0)),
            pl.BlockSpec((D, D), lambda i: (0, 0)),
            pl.BlockSpec((1, D), lambda i: (0, 0)),
        ],
        out_specs=pl.BlockSpec((BLK, D), lambda i: (i, 0)),
        out_shape=jax.ShapeDtypeStruct((N, D), jnp.float32),
    )(g1, part, dis, w2p, b1p)


def _tc3(g2, part, dis, b2p, wlp, blp, batch3):
    return pl.pallas_call(
        _tc3_body,
        grid=(NBLK,),
        in_specs=[
            pl.BlockSpec((BLK, D), lambda i: (i, 0)),
            pl.BlockSpec((NC, BLK, D), lambda i: (0, i, 0)),
            pl.BlockSpec((BLK, 1), lambda i: (i, 0)),
            pl.BlockSpec((1, D), lambda i: (0, 0)),
            pl.BlockSpec((D, D), lambda i: (0, 0)),
            pl.BlockSpec((1, 1), lambda i: (0, 0)),
            pl.BlockSpec((1, 1, BLK), lambda i: (i, 0, 0)),
        ],
        out_specs=pl.BlockSpec((G, 1), lambda i: (0, 0)),
        out_shape=jax.ShapeDtypeStruct((G, 1), jnp.float32),
        scratch_shapes=[
            pltpu.VMEM((G, D), jnp.float32),
            pltpu.VMEM((G, D), jnp.float32),
        ],
    )(g2, part, dis, b2p, wlp, blp, batch3)


def kernel(x, edge_index, batch, W1, b1, W2, b2, Wl, bl):
    src = edge_index[0].astype(jnp.int32)
    dst = edge_index[1].astype(jnp.int32)
    pad = EPAD - E
    srcp = jnp.concatenate([src, jnp.zeros((pad,), jnp.int32)])
    dstp = jnp.concatenate([dst, jnp.full((pad,), NPAD - 1, jnp.int32)])
    eidx = jnp.stack([srcp.reshape(NW, NCHUNK, CH),
                      dstp.reshape(NW, NCHUNK, CH)], axis=2)
    batch3 = batch.astype(jnp.int32).reshape(NBLK, 1, BLK)

    H = W1.shape[1]
    w1p = jnp.pad(W1, ((0, 0), (0, D - H)))
    w2p = jnp.pad(W2, ((0, D - H), (0, D - H)))
    b1p = jnp.pad(b1, (0, D - H)).reshape(1, D)
    b2p = jnp.pad(b2, (0, D - H)).reshape(1, D)
    wlp = jnp.pad(Wl, ((0, D - H), (0, D - 1)))
    blp = bl.reshape(1, 1)

    zerosD = jnp.zeros((RCH, D), jnp.float32)
    onesD = jnp.tile(jnp.eye(1, D, 0, dtype=jnp.float32), (CH, 1))

    degp = _deg_kernel()(eidx, onesD, zerosD)
    g1, dis = _tc1(x, w1p, degp)
    s1 = _feat_kernel()(g1, eidx, zerosD)
    g2 = _tc2(g1, s1, dis, w2p, b1p)
    s2 = _feat_kernel()(g2, eidx, zerosD)
    return _tc3(g2, s2, dis, b2p, wlp, blp, batch3)


# speedup vs baseline: 1.1235x; 1.0231x over previous
import functools

import jax
import jax.numpy as jnp
from jax import lax
from jax.experimental import pallas as pl
from jax.experimental.pallas import tpu as pltpu
from jax.experimental.pallas import tpu_sc as plsc

N = 10000
E = 320000
D = 128
G = 64
NC, NS, L = 2, 16, 16
NW = NC * NS
NPAD = 10240
EPW = NPAD
EPAD = NW * EPW
CH = 128
NCHUNK = EPW // CH
RCH = 128
CPS = NPAD // RCH // NS
BLK = 1000
NBLK = N // BLK
W16 = 16

def _mesh():
    return plsc.VectorSubcoreMesh(
        core_axis_name="c", subcore_axis_name="s", num_cores=NC, num_subcores=NS)


NBUF = 2
SEGF = 20
NSEGF = NCHUNK // SEGF
ZCH = NPAD // CH // NS


def _feat_body(g_hbm, eidx_hbm, zeros_hbm, part_hbm, acc, ebuf, rows,
               gs0, gs1, ss0, ss1):
    gs = (gs0, gs1)
    ss = (ss0, ss1)
    c = lax.axis_index("c")
    s = lax.axis_index("s")
    wid = s * NC + c

    pltpu.sync_copy(zeros_hbm, rows.at[0])
    for k in range(ZCH):
        row0 = (s * ZCH + k) * CH
        pltpu.sync_copy(rows.at[0], acc.at[pl.ds(row0, CH)])
    plsc.subcore_barrier()

    @pl.loop(0, NSEGF)
    def _(seg):
        pltpu.sync_copy(eidx_hbm.at[wid, pl.ds(seg * SEGF, SEGF)], ebuf)
        for b in range(NBUF):
            pltpu.async_copy(g_hbm.at[ebuf.at[b, 0]], rows.at[b], gs[b])

        @pl.loop(0, SEGF, step=NBUF)
        def _(q):
            for b in range(NBUF):
                j = q + b
                pltpu.make_async_copy(g_hbm.at[ebuf.at[j, 0]], rows.at[b], gs[b]).wait()
                pltpu.async_copy(rows.at[b], acc.at[ebuf.at[j, 1]], ss[b], add=True)

                @pl.when(j + NBUF < SEGF)
                def _():
                    pltpu.make_async_copy(rows.at[b], acc.at[ebuf.at[j, 1]], ss[b]).wait()
                    pltpu.async_copy(g_hbm.at[ebuf.at[j + NBUF, 0]], rows.at[b], gs[b])

        for b in range(NBUF):
            pltpu.make_async_copy(rows.at[b], acc.at[ebuf.at[0, 1]], ss[b]).wait()

    plsc.subcore_barrier()
    for k in range(CPS):
        row0 = (s * CPS + k) * RCH
        pltpu.sync_copy(acc.at[pl.ds(row0, RCH)], part_hbm.at[c, pl.ds(row0, RCH)])


NDBUF = 4


def _deg_body(eidx_hbm, ones_hbm, zeros_hbm, part_hbm, acc, ebuf, ones_v,
              ss0, ss1, ss2, ss3):
    ss = (ss0, ss1, ss2, ss3)
    c = lax.axis_index("c")
    s = lax.axis_index("s")
    wid = s * NC + c

    pltpu.sync_copy(eidx_hbm.at[wid], ebuf)
    pltpu.sync_copy(zeros_hbm, ones_v)
    for k in range(CPS):
        row0 = (s * CPS + k) * RCH
        pltpu.sync_copy(ones_v, acc.at[pl.ds(row0, RCH)])
    pltpu.sync_copy(ones_hbm, ones_v)
    plsc.subcore_barrier()

    @pl.loop(0, NCHUNK, step=NDBUF)
    def _(g):
        for b in range(NDBUF):
            i = g + b

            @pl.when(g >= NDBUF)
            def _():
                pltpu.make_async_copy(ones_v, acc.at[ebuf.at[i, 1]], ss[b]).wait()

            pltpu.async_copy(ones_v, acc.at[ebuf.at[i, 1]], ss[b], add=True)

    for b in range(NDBUF):
        i = NCHUNK - NDBUF + b
        pltpu.make_async_copy(ones_v, acc.at[ebuf.at[i, 1]], ss[b]).wait()

    plsc.subcore_barrier()
    for k in range(CPS):
        row0 = (s * CPS + k) * RCH
        pltpu.sync_copy(acc.at[pl.ds(row0, RCH)], part_hbm.at[c, pl.ds(row0, RCH)])


@functools.cache
def _feat_kernel():
    return pl.kernel(
        _feat_body,
        out_type=jax.ShapeDtypeStruct((NC, NPAD, D), jnp.float32),
        mesh=_mesh(),
        scratch_types=[
            pltpu.VMEM_SHARED((NPAD, D), jnp.float32),
            pltpu.VMEM((SEGF, 2, CH), jnp.int32),
            pltpu.VMEM((NBUF, CH, D), jnp.float32),
        ] + [pltpu.SemaphoreType.DMA] * (2 * NBUF),
    )


@functools.cache
def _deg_kernel():
    return pl.kernel(
        _deg_body,
        out_type=jax.ShapeDtypeStruct((NC, NPAD, D), jnp.float32),
        mesh=_mesh(),
        scratch_types=[
            pltpu.VMEM_SHARED((NPAD, D), jnp.float32),
            pltpu.VMEM((NCHUNK, 2, CH), jnp.int32),
            pltpu.VMEM((CH, D), jnp.float32),
        ] + [pltpu.SemaphoreType.DMA] * NDBUF,
    )


def _tc1_body(x_ref, w_ref, degp_ref, g_ref, dis_ref):
    deg = degp_ref[0, :, 0:1] + degp_ref[1, :, 0:1] + 1.0
    dis = lax.rsqrt(deg)
    h = jnp.dot(x_ref[...], w_ref[...], preferred_element_type=jnp.float32)
    g_ref[...] = h * dis
    dis_ref[...] = dis


def _tc2_body(g1_ref, part_ref, dis_ref, w2_ref, b1_ref, g2_ref):
    stot = part_ref[0] + part_ref[1]
    dis = dis_ref[...]
    z = dis * (stot + g1_ref[...]) + b1_ref[...]
    h = jnp.tanh(z)
    g2_ref[...] = jnp.dot(h, w2_ref[...], preferred_element_type=jnp.float32) * dis


def _tc3_body(g2_ref, part_ref, dis_ref, b2_ref, wl_ref, bl_ref, batch_ref,
              out_ref, pool_acc, cnt_acc):
    i = pl.program_id(0)
    stot = part_ref[0] + part_ref[1]
    dis = dis_ref[...]
    z = dis * (stot + g2_ref[...]) + b2_ref[...]
    b = batch_ref[0]
    iot = lax.broadcasted_iota(jnp.int32, (G, BLK), 0)
    oh = (b == iot).astype(jnp.float32)

    @pl.when(i == 0)
    def _():
        pool_acc[...] = jnp.zeros_like(pool_acc)
        cnt_acc[...] = jnp.zeros_like(cnt_acc)

    pool_acc[...] += jnp.dot(oh, z, preferred_element_type=jnp.float32)
    cnt_acc[...] += jnp.broadcast_to(
        jnp.sum(oh, axis=1, keepdims=True), (G, D))

    @pl.when(i == NBLK - 1)
    def _():
        pooled = pool_acc[...] / jnp.maximum(cnt_acc[...], 1.0)
        o = jnp.dot(pooled, wl_ref[...], preferred_element_type=jnp.float32)
        out_ref[...] = o[:, 0:1] + bl_ref[...]


def _tc1(x, w1p, degp):
    return pl.pallas_call(
        _tc1_body,
        grid=(NBLK,),
        in_specs=[
            pl.BlockSpec((BLK, D), lambda i: (i, 0)),
            pl.BlockSpec((D, D), lambda i: (0, 0)),
            pl.BlockSpec((NC, BLK, D), lambda i: (0, i, 0)),
        ],
        out_specs=[
            pl.BlockSpec((BLK, D), lambda i: (i, 0)),
            pl.BlockSpec((BLK, 1), lambda i: (i, 0)),
        ],
        out_shape=[
            jax.ShapeDtypeStruct((N, D), jnp.float32),
            jax.ShapeDtypeStruct((N, 1), jnp.float32),
        ],
    )(x, w1p, degp)


def _tc2(g1, part, dis, w2p, b1p):
    return pl.pallas_call(
        _tc2_body,
        grid=(NBLK,),
        in_specs=[
            pl.BlockSpec((BLK, D), lambda i: (i, 0)),
            pl.BlockSpec((NC, BLK, D), lambda i: (0, i, 0)),
            pl.BlockSpec((BLK, 1), lambda i: (i, 0)),
            pl.BlockSpec((D, D), lambda i: (0, 0)),
            pl.BlockSpec((1, D), lambda i: (0, 0)),
        ],
        out_specs=pl.BlockSpec((BLK, D), lambda i: (i, 0)),
        out_shape=jax.ShapeDtypeStruct((N, D), jnp.float32),
    )(g1, part, dis, w2p, b1p)


def _tc3(g2, part, dis, b2p, wlp, blp, batch3):
    return pl.pallas_call(
        _tc3_body,
        grid=(NBLK,),
        in_specs=[
            pl.BlockSpec((BLK, D), lambda i: (i, 0)),
            pl.BlockSpec((NC, BLK, D), lambda i: (0, i, 0)),
            pl.BlockSpec((BLK, 1), lambda i: (i, 0)),
            pl.BlockSpec((1, D), lambda i: (0, 0)),
            pl.BlockSpec((D, D), lambda i: (0, 0)),
            pl.BlockSpec((1, 1), lambda i: (0, 0)),
            pl.BlockSpec((1, 1, BLK), lambda i: (i, 0, 0)),
        ],
        out_specs=pl.BlockSpec((G, 1), lambda i: (0, 0)),
        out_shape=jax.ShapeDtypeStruct((G, 1), jnp.float32),
        scratch_shapes=[
            pltpu.VMEM((G, D), jnp.float32),
            pltpu.VMEM((G, D), jnp.float32),
        ],
    )(g2, part, dis, b2p, wlp, blp, batch3)


def kernel(x, edge_index, batch, W1, b1, W2, b2, Wl, bl):
    src = edge_index[0].astype(jnp.int32)
    dst = edge_index[1].astype(jnp.int32)
    pad = EPAD - E
    srcp = jnp.concatenate([src, jnp.zeros((pad,), jnp.int32)])
    dstp = jnp.concatenate([dst, jnp.full((pad,), NPAD - 1, jnp.int32)])
    eidx = jnp.stack([srcp.reshape(NCHUNK, NW, CH).transpose(1, 0, 2),
                      dstp.reshape(NCHUNK, NW, CH).transpose(1, 0, 2)], axis=2)
    batch3 = batch.astype(jnp.int32).reshape(NBLK, 1, BLK)

    H = W1.shape[1]
    w1p = jnp.pad(W1, ((0, 0), (0, D - H)))
    w2p = jnp.pad(W2, ((0, D - H), (0, D - H)))
    b1p = jnp.pad(b1, (0, D - H)).reshape(1, D)
    b2p = jnp.pad(b2, (0, D - H)).reshape(1, D)
    wlp = jnp.pad(Wl, ((0, D - H), (0, D - 1)))
    blp = bl.reshape(1, 1)

    zerosD = jnp.zeros((RCH, D), jnp.float32)
    onesD = jnp.tile(jnp.eye(1, D, 0, dtype=jnp.float32), (CH, 1))

    degp = _deg_kernel()(eidx, onesD, zerosD)
    g1, dis = _tc1(x, w1p, degp)
    s1 = _feat_kernel()(g1, eidx, zerosD)
    g2 = _tc2(g1, s1, dis, w2p, b1p)
    s2 = _feat_kernel()(g2, eidx, zerosD)
    return _tc3(g2, s2, dis, b2p, wlp, blp, batch3)


# speedup vs baseline: 1.1755x; 1.0463x over previous
import functools

import jax
import jax.numpy as jnp
from jax import lax
from jax.experimental import pallas as pl
from jax.experimental.pallas import tpu as pltpu
from jax.experimental.pallas import tpu_sc as plsc

N = 10000
E = 320000
D = 128
G = 64
NC, NS, L = 2, 16, 16
NW = NC * NS
NPAD = 10240
EPW = NPAD
EPAD = NW * EPW
CH = 128
NCHUNK = EPW // CH
RCH = 128
CPS = NPAD // RCH // NS
BLK = 1000
NBLK = N // BLK
W16 = 16

def _mesh():
    return plsc.VectorSubcoreMesh(
        core_axis_name="c", subcore_axis_name="s", num_cores=NC, num_subcores=NS)


NBUF = 2
SEGF = 20
NSEGF = NCHUNK // SEGF
ZCH = NPAD // CH // NS


def _feat_body(g_hbm, eidx_hbm, zeros_hbm, part_hbm, acc, ebuf, rows,
               gs0, gs1, ss0, ss1):
    gs = (gs0, gs1)
    ss = (ss0, ss1)
    c = lax.axis_index("c")
    s = lax.axis_index("s")
    wid = s * NC + c

    pltpu.sync_copy(zeros_hbm, rows.at[0])
    for k in range(ZCH):
        row0 = (s * ZCH + k) * CH
        pltpu.sync_copy(rows.at[0], acc.at[pl.ds(row0, CH)])
    plsc.subcore_barrier()

    @pl.loop(0, NSEGF)
    def _(seg):
        pltpu.sync_copy(eidx_hbm.at[wid, pl.ds(seg * SEGF, SEGF)], ebuf)
        for b in range(NBUF):
            pltpu.async_copy(g_hbm.at[ebuf.at[b, 0]], rows.at[b], gs[b])

        @pl.loop(0, SEGF, step=NBUF)
        def _(q):
            for b in range(NBUF):
                j = q + b
                pltpu.make_async_copy(g_hbm.at[ebuf.at[j, 0]], rows.at[b], gs[b]).wait()
                pltpu.async_copy(rows.at[b], acc.at[ebuf.at[j, 1]], ss[b], add=True)

                @pl.when(j + NBUF < SEGF)
                def _():
                    pltpu.make_async_copy(rows.at[b], acc.at[ebuf.at[j, 1]], ss[b]).wait()
                    pltpu.async_copy(g_hbm.at[ebuf.at[j + NBUF, 0]], rows.at[b], gs[b])

        for b in range(NBUF):
            pltpu.make_async_copy(rows.at[b], acc.at[ebuf.at[0, 1]], ss[b]).wait()

    plsc.subcore_barrier()
    for k in range(CPS):
        row0 = (s * CPS + k) * RCH
        pltpu.sync_copy(acc.at[pl.ds(row0, RCH)], part_hbm.at[c, pl.ds(row0, RCH)])


NDBUF = 4


def _deg_body(eidx_hbm, ones_hbm, zeros_hbm, part_hbm, acc, ebuf, ones_v,
              ss0, ss1, ss2, ss3):
    ss = (ss0, ss1, ss2, ss3)
    c = lax.axis_index("c")
    s = lax.axis_index("s")
    wid = s * NC + c

    pltpu.sync_copy(eidx_hbm.at[wid], ebuf)
    pltpu.sync_copy(zeros_hbm, ones_v)
    for k in range(CPS):
        row0 = (s * CPS + k) * RCH
        pltpu.sync_copy(ones_v, acc.at[pl.ds(row0, RCH)])
    pltpu.sync_copy(ones_hbm, ones_v)
    plsc.subcore_barrier()

    @pl.loop(0, NCHUNK, step=NDBUF)
    def _(g):
        for b in range(NDBUF):
            i = g + b

            @pl.when(g >= NDBUF)
            def _():
                pltpu.make_async_copy(ones_v, acc.at[ebuf.at[i, 1]], ss[b]).wait()

            pltpu.async_copy(ones_v, acc.at[ebuf.at[i, 1]], ss[b], add=True)

    for b in range(NDBUF):
        i = NCHUNK - NDBUF + b
        pltpu.make_async_copy(ones_v, acc.at[ebuf.at[i, 1]], ss[b]).wait()

    plsc.subcore_barrier()
    for k in range(CPS):
        row0 = (s * CPS + k) * RCH
        pltpu.sync_copy(acc.at[pl.ds(row0, RCH)], part_hbm.at[c, pl.ds(row0, RCH)])


@functools.cache
def _feat_kernel():
    return pl.kernel(
        _feat_body,
        out_type=jax.ShapeDtypeStruct((NC, NPAD, D), jnp.float32),
        mesh=_mesh(),
        scratch_types=[
            pltpu.VMEM_SHARED((NPAD, D), jnp.float32),
            pltpu.VMEM((SEGF, 2, CH), jnp.int32),
            pltpu.VMEM((NBUF, CH, D), jnp.float32),
        ] + [pltpu.SemaphoreType.DMA] * (2 * NBUF),
    )


@functools.cache
def _deg_kernel():
    return pl.kernel(
        _deg_body,
        out_type=jax.ShapeDtypeStruct((NC, NPAD, D), jnp.float32),
        mesh=_mesh(),
        scratch_types=[
            pltpu.VMEM_SHARED((NPAD, D), jnp.float32),
            pltpu.VMEM((NCHUNK, 2, CH), jnp.int32),
            pltpu.VMEM((CH, D), jnp.float32),
        ] + [pltpu.SemaphoreType.DMA] * NDBUF,
    )


def _tc1_body(x_ref, w_ref, degp_ref, g_ref, dis_ref):
    deg = degp_ref[0, :, 0:1] + degp_ref[1, :, 0:1] + 1.0
    dis = lax.rsqrt(deg)
    h = jnp.dot(x_ref[...], w_ref[...], preferred_element_type=jnp.float32)
    g_ref[...] = h * dis
    dis_ref[...] = dis


def _tc2_body(g1_ref, part_ref, dis_ref, w2_ref, b1_ref, g2_ref):
    stot = part_ref[0] + part_ref[1]
    dis = dis_ref[...]
    z = dis * (stot + g1_ref[...]) + b1_ref[...]
    h = jnp.tanh(z)
    g2_ref[...] = jnp.dot(h, w2_ref[...], preferred_element_type=jnp.float32) * dis


def _tc3_body(g2_ref, part_ref, dis_ref, b2_ref, wl_ref, bl_ref, batch_ref,
              out_ref, pool_acc, cnt_acc):
    i = pl.program_id(0)
    stot = part_ref[0] + part_ref[1]
    dis = dis_ref[...]
    z = dis * (stot + g2_ref[...]) + b2_ref[...]
    b = batch_ref[0]
    iot = lax.broadcasted_iota(jnp.int32, (G, BLK), 0)
    oh = (b == iot).astype(jnp.float32)

    @pl.when(i == 0)
    def _():
        pool_acc[...] = jnp.zeros_like(pool_acc)
        cnt_acc[...] = jnp.zeros_like(cnt_acc)

    pool_acc[...] += jnp.dot(oh, z, preferred_element_type=jnp.float32)
    cnt_acc[...] += jnp.broadcast_to(
        jnp.sum(oh, axis=1, keepdims=True), (G, D))

    @pl.when(i == NBLK - 1)
    def _():
        pooled = pool_acc[...] / jnp.maximum(cnt_acc[...], 1.0)
        o = jnp.dot(pooled, wl_ref[...], preferred_element_type=jnp.float32)
        out_ref[...] = o[:, 0:1] + bl_ref[...]


def _tc1(x, w1p, degp):
    return pl.pallas_call(
        _tc1_body,
        grid=(NBLK,),
        in_specs=[
            pl.BlockSpec((BLK, D), lambda i: (i, 0)),
            pl.BlockSpec((D, D), lambda i: (0, 0)),
            pl.BlockSpec((NC, BLK, D), lambda i: (0, i, 0)),
        ],
        out_specs=[
            pl.BlockSpec((BLK, D), lambda i: (i, 0)),
            pl.BlockSpec((BLK, 1), lambda i: (i, 0)),
        ],
        out_shape=[
            jax.ShapeDtypeStruct((N, D), jnp.float32),
            jax.ShapeDtypeStruct((N, 1), jnp.float32),
        ],
    )(x, w1p, degp)


def _tc2(g1, part, dis, w2p, b1p):
    return pl.pallas_call(
        _tc2_body,
        grid=(NBLK,),
        in_specs=[
            pl.BlockSpec((BLK, D), lambda i: (i, 0)),
            pl.BlockSpec((NC, BLK, D), lambda i: (0, i, 0)),
            pl.BlockSpec((BLK, 1), lambda i: (i, 0)),
            pl.BlockSpec((D, D), lambda i: (0, 0)),
            pl.BlockSpec((1, D), lambda i: (0, 0)),
        ],
        out_specs=pl.BlockSpec((BLK, D), lambda i: (i, 0)),
        out_shape=jax.ShapeDtypeStruct((N, D), jnp.float32),
    )(g1, part, dis, w2p, b1p)


def _tc3(g2, part, dis, b2p, wlp, blp, batch3):
    return pl.pallas_call(
        _tc3_body,
        grid=(NBLK,),
        in_specs=[
            pl.BlockSpec((BLK, D), lambda i: (i, 0)),
            pl.BlockSpec((NC, BLK, D), lambda i: (0, i, 0)),
            pl.BlockSpec((BLK, 1), lambda i: (i, 0)),
            pl.BlockSpec((1, D), lambda i: (0, 0)),
            pl.BlockSpec((D, D), lambda i: (0, 0)),
            pl.BlockSpec((1, 1), lambda i: (0, 0)),
            pl.BlockSpec((1, 1, BLK), lambda i: (i, 0, 0)),
        ],
        out_specs=pl.BlockSpec((G, 1), lambda i: (0, 0)),
        out_shape=jax.ShapeDtypeStruct((G, 1), jnp.float32),
        scratch_shapes=[
            pltpu.VMEM((G, D), jnp.float32),
            pltpu.VMEM((G, D), jnp.float32),
        ],
    )(g2, part, dis, b2p, wlp, blp, batch3)


def kernel(x, edge_index, batch, W1, b1, W2, b2, Wl, bl):
    src = edge_index[0].astype(jnp.int32)
    dst = edge_index[1].astype(jnp.int32)
    pad = EPAD - E
    srcp = jnp.concatenate([src, jnp.zeros((pad,), jnp.int32)])
    dstp = jnp.concatenate([dst, jnp.full((pad,), NPAD - 1, jnp.int32)])
    eidx = jnp.stack([srcp.reshape(CH, NW, NCHUNK).transpose(1, 2, 0),
                      dstp.reshape(CH, NW, NCHUNK).transpose(1, 2, 0)], axis=2)
    batch3 = batch.astype(jnp.int32).reshape(NBLK, 1, BLK)

    H = W1.shape[1]
    w1p = jnp.pad(W1, ((0, 0), (0, D - H)))
    w2p = jnp.pad(W2, ((0, D - H), (0, D - H)))
    b1p = jnp.pad(b1, (0, D - H)).reshape(1, D)
    b2p = jnp.pad(b2, (0, D - H)).reshape(1, D)
    wlp = jnp.pad(Wl, ((0, D - H), (0, D - 1)))
    blp = bl.reshape(1, 1)

    zerosD = jnp.zeros((RCH, D), jnp.float32)
    onesD = jnp.tile(jnp.eye(1, D, 0, dtype=jnp.float32), (CH, 1))

    degp = _deg_kernel()(eidx, onesD, zerosD)
    g1, dis = _tc1(x, w1p, degp)
    s1 = _feat_kernel()(g1, eidx, zerosD)
    g2 = _tc2(g1, s1, dis, w2p, b1p)
    s2 = _feat_kernel()(g2, eidx, zerosD)
    return _tc3(g2, s2, dis, b2p, wlp, blp, batch3)
